# Initial kernel scaffold; baseline (speedup 1.0000x reference)
import jax
import jax.numpy as jnp
from jax.experimental import pallas as pl


def kernel(x, edge_index, edge_attr, We, be, W1, b1, W2, b2):
    raise NotImplementedError("write your pallas kernel here")


# speedup vs baseline: 2.2773x; 2.2773x over previous
import functools

import jax
import jax.numpy as jnp
from jax import lax
from jax.experimental import pallas as pl
from jax.experimental.pallas import tpu as pltpu
from jax.experimental.pallas import tpu_sc as plsc

N_NODES = 10000
N_EDGES = 320000
D = 128
D_EDGE = 16

NC = 2
NS = 16
N_TILES = NC * NS

E_PER_CORE = N_EDGES // NC
E_PER_TILE = E_PER_CORE // NS
CHUNK = 80
N_CHUNKS = E_PER_TILE // CHUNK
ZROWS = 200
N_ZCH = N_NODES // ZROWS


def _emb_body(attr_ref, we_ref, be_ref, out_ref):
    out_ref[...] = (
        jnp.dot(attr_ref[...], we_ref[...], preferred_element_type=jnp.float32)
        + be_ref[...]
    )


def _edge_embedding(edge_attr, We, be2d):
    blk = 2000
    return pl.pallas_call(
        _emb_body,
        grid=(N_EDGES // blk,),
        in_specs=[
            pl.BlockSpec((blk, D_EDGE), lambda i: (i, 0)),
            pl.BlockSpec((D_EDGE, D), lambda i: (0, 0)),
            pl.BlockSpec((1, D), lambda i: (0, 0)),
        ],
        out_specs=pl.BlockSpec((blk, D), lambda i: (i, 0)),
        out_shape=jax.ShapeDtypeStruct((N_EDGES, D), jnp.float32),
    )(edge_attr, We, be2d)


def _sc_body(x_hbm, row_hbm, col_hbm, emb_hbm, out_hbm,
             row_v, col_v, msg_v, xg_v, zero_v, agg_sh, gsem):
    c = lax.axis_index("c")
    s = lax.axis_index("s")

    def _zero_row(r, _):
        for j in range(D // 16):
            zero_v[r, pl.ds(j * 16, 16)] = jnp.zeros((16,), jnp.float32)
        return 0

    lax.fori_loop(0, ZROWS, _zero_row, 0)
    for t in range(-(-N_ZCH // NS)):
        idx = s + t * NS

        @pl.when(idx < N_ZCH)
        def _():
            pltpu.sync_copy(zero_v, agg_sh.at[pl.ds(idx * ZROWS, ZROWS)])

    plsc.subcore_barrier()

    base_e = c * E_PER_CORE + s * E_PER_TILE

    def _chunk(g, _):
        e0 = base_e + g * CHUNK
        pltpu.sync_copy(row_hbm.at[pl.ds(e0, CHUNK)], row_v)
        pltpu.sync_copy(col_hbm.at[pl.ds(e0, CHUNK)], col_v)
        pltpu.sync_copy(emb_hbm.at[pl.ds(e0, CHUNK)], msg_v)
        pltpu.async_copy(x_hbm.at[row_v], xg_v, gsem).wait()

        def _relu_row(r, _):
            for j in range(D // 16):
                a = msg_v[r, pl.ds(j * 16, 16)]
                b = xg_v[r, pl.ds(j * 16, 16)]
                msg_v[r, pl.ds(j * 16, 16)] = jnp.maximum(a + b, 0.0)
            return 0

        lax.fori_loop(0, CHUNK, _relu_row, 0)
        pltpu.sync_copy(msg_v, agg_sh.at[col_v], add=True)
        return 0

    lax.fori_loop(0, N_CHUNKS, _chunk, 0)
    plsc.subcore_barrier()

    for t in range(-(-N_ZCH // NS)):
        idx = s + t * NS

        @pl.when(idx < N_ZCH)
        def _():
            r0 = idx * ZROWS
            pltpu.sync_copy(agg_sh.at[pl.ds(r0, ZROWS)], out_hbm.at[c, pl.ds(r0, ZROWS)])


def _sc_aggregate(x, row, col, emb):
    mesh = plsc.VectorSubcoreMesh(
        core_axis_name="c", subcore_axis_name="s", num_cores=NC, num_subcores=NS
    )
    fn = functools.partial(
        pl.kernel,
        out_type=jax.ShapeDtypeStruct((NC, N_NODES, D), jnp.float32),
        mesh=mesh,
        scratch_types=[
            pltpu.VMEM((CHUNK,), jnp.int32),
            pltpu.VMEM((CHUNK,), jnp.int32),
            pltpu.VMEM((CHUNK, D), jnp.float32),
            pltpu.VMEM((CHUNK, D), jnp.float32),
            pltpu.VMEM((ZROWS, D), jnp.float32),
            pltpu.VMEM_SHARED((N_NODES, D), jnp.float32),
            pltpu.SemaphoreType.DMA,
        ],
    )(_sc_body)
    return fn(x, row, col, emb)


def _mlp_body(x_ref, a0_ref, a1_ref, w1_ref, b1_ref, w2_ref, b2_ref, out_ref):
    h = x_ref[...] + a0_ref[0] + a1_ref[0]
    h = jnp.maximum(
        jnp.dot(h, w1_ref[...], preferred_element_type=jnp.float32) + b1_ref[...],
        0.0,
    )
    out_ref[...] = (
        jnp.dot(h, w2_ref[...], preferred_element_type=jnp.float32) + b2_ref[...]
    )


def _mlp(x, agg, W1, b1_2d, W2, b2_2d):
    blk = 1000
    return pl.pallas_call(
        _mlp_body,
        grid=(N_NODES // blk,),
        in_specs=[
            pl.BlockSpec((blk, D), lambda i: (i, 0)),
            pl.BlockSpec((1, blk, D), lambda i: (0, i, 0)),
            pl.BlockSpec((1, blk, D), lambda i: (1, i, 0)),
            pl.BlockSpec((D, D), lambda i: (0, 0)),
            pl.BlockSpec((1, D), lambda i: (0, 0)),
            pl.BlockSpec((D, D), lambda i: (0, 0)),
            pl.BlockSpec((1, D), lambda i: (0, 0)),
        ],
        out_specs=pl.BlockSpec((blk, D), lambda i: (i, 0)),
        out_shape=jax.ShapeDtypeStruct((N_NODES, D), jnp.float32),
    )(x, agg, agg, W1, b1_2d, W2, b2_2d)


def kernel(x, edge_index, edge_attr, We, be, W1, b1, W2, b2):
    row = edge_index[0].astype(jnp.int32)
    col = edge_index[1].astype(jnp.int32)
    emb = _edge_embedding(edge_attr, We, be.reshape(1, D))
    agg = _sc_aggregate(x, row, col, emb)
    return _mlp(x, agg, W1, b1.reshape(1, D), W2, b2.reshape(1, D))
